# baseline (speedup 1.0000x reference)
import functools

import jax
import jax.numpy as jnp
import numpy as np
from jax import lax
from jax.experimental import pallas as pl
from jax.experimental.pallas import tpu as pltpu
from jax.experimental.pallas import tpu_sc as plsc

H = 224
W = 224
C = 96
HO = 3 * H
WO = 3 * W
L = 16
NC = 2
NS = 16
NW = NC * NS
CPW = C // NW
CH = 12
NCHUNK = HO // CH
IPC = CH // 3
G = WO // L
NBUF = 2


def _col_index_table() -> np.ndarray:
    q = np.arange(WO)
    j = q // 3 + q % 3 - 1
    j = np.where(j < 0, 1, np.where(j > W - 1, W - 2, j))
    return j.astype(np.int32)


def _dc_body(x_hbm, cidx_hbm, lscale_hbm, out_hbm,
             xin, bufs, idx_tab, last_scale, *sems):
    wid = lax.axis_index("s") * NC + lax.axis_index("c")

    pltpu.sync_copy(cidx_hbm, idx_tab)
    pltpu.sync_copy(lscale_hbm, last_scale)

    def _out_copy(b, ci, ch, sem):
        return pltpu.make_async_copy(
            bufs.at[b], out_hbm.at[0, ch, pl.ds(ci * CH, CH)], sem)

    for k in range(CPW):
        ch = wid * CPW + k
        pltpu.sync_copy(x_hbm.at[0, ch], xin)

        @pl.loop(0, NCHUNK, step=NBUF)
        def _chunks(ci0):
            for b in range(NBUF):
                ci = ci0 + b
                sem = sems[b]
                buf = bufs.at[b]

                @pl.when(ci >= NBUF)
                def _():
                    _out_copy(b, ci - NBUF, ch, sem).wait()

                bases = []
                for pl_ in range(IPC + 2):
                    p = ci * IPC + pl_
                    ir = jnp.where(p == 0, 1,
                                   jnp.where(p == H + 1, H - 2, p - 1))
                    bases.append(jnp.broadcast_to(ir, (L,)))

                for g in range(G):
                    cidx = idx_tab[pl.ds(g * L, L)]
                    vals = [plsc.load_gather(xin, [bv, cidx]) for bv in bases]
                    if g == G - 1:
                        ls = last_scale[...]
                        vals = [v * ls for v in vals]
                    for rloc in range(CH):
                        buf[rloc, pl.ds(g * L, L)] = \
                            vals[rloc // 3 + rloc % 3]

                @pl.when(ci == NCHUNK - 1)
                def _():
                    for g in range(G):
                        w = buf[CH - 1, pl.ds(g * L, L)]
                        buf[CH - 1, pl.ds(g * L, L)] = w + w

                _out_copy(b, ci, ch, sem).start()

        for b in range(NBUF):
            ci = NCHUNK - NBUF + b
            _out_copy(b, ci, ch, sems[b]).wait()


@functools.cache
def _dc_kernel():
    return pl.kernel(
        _dc_body,
        out_type=jax.ShapeDtypeStruct((1, C, HO, WO), jnp.float32),
        compiler_params=pltpu.CompilerParams(
            use_tc_tiling_on_sc=False, needs_layout_passes=False),
        mesh=plsc.VectorSubcoreMesh(
            core_axis_name="c", subcore_axis_name="s",
            num_cores=NC, num_subcores=NS,
        ),
        scratch_types=[
            pltpu.VMEM((H, W), jnp.float32),
            pltpu.VMEM((NBUF, CH, WO), jnp.float32),
            pltpu.VMEM((WO,), jnp.int32),
            pltpu.VMEM((L,), jnp.float32),
        ] + [pltpu.SemaphoreType.DMA] * NBUF,
    )


def kernel(x):
    cidx = jnp.asarray(_col_index_table())
    lscale = jnp.asarray(
        np.where(np.arange(L) == L - 1, 2.0, 1.0).astype(np.float32))
    return _dc_kernel()(x, cidx, lscale)
